# Initial kernel scaffold; baseline (speedup 1.0000x reference)
import jax
import jax.numpy as jnp
from jax.experimental import pallas as pl


def kernel(x, edge_index, edge_attr, batch, We, be, Wpre, bpre, Wpost, bpost, Wlin, blin, gamma, beta, Wm1, bm1, Wm2, bm2, Wm3, bm3):
    raise NotImplementedError("write your pallas kernel here")


# speedup vs baseline: 8.3614x; 8.3614x over previous
import functools

import jax
import jax.numpy as jnp
import numpy as np
from jax import lax
from jax.experimental import pallas as pl
from jax.experimental.pallas import tpu as pltpu
from jax.experimental.pallas import tpu_sc as plsc

N_NODES = 10000
N_EDGES = 160000
F = 80
T = 5
F_OUT = F // T
EDGE_DIM = 16
N_LAYERS = 2
N_GRAPHS = 16

_deg_hist = np.array([0, 0, 0, 0, 0, 0, 0, 0, 0, 0, 50, 150, 400, 800, 1200, 1500, 1500, 1200, 800, 400, 150, 50, 0, 0, 0], dtype=np.float32)
_bins = np.arange(_deg_hist.shape[0], dtype=np.float32)
_AVG_DEG_LOG = float((np.log(_bins + 1.0) * _deg_hist).sum() / _deg_hist.sum())

NW = 32
OWN = 320
N_PAD = NW * OWN
CAP = 6400
CH = 128
NCH = CAP // CH
EPT = N_EDGES // NW
GFULL = EPT // CH
GREM = EPT - GFULL * CH
SCAN = 2000
NSCAN = N_EDGES // SCAN

_sc_mesh = plsc.VectorSubcoreMesh(core_axis_name="c", subcore_axis_name="s")
_sc_params = pltpu.CompilerParams(use_tc_tiling_on_sc=False)


def _wid():
    return lax.axis_index("s") * 2 + lax.axis_index("c")


def _sread(ref, i):
    return ref[pl.ds(i, 16)][0]


def _prep_body(dst_hbm, eid_hbm, dstl_hbm, deg_hbm,
               dstv, csr_eid, csr_dstl, degv, counts, sem):
    w = _wid()
    lo = w * OWN
    z16 = jnp.zeros((16,), jnp.int32)

    def zc(i, _):
        counts[i] = 0
        return ()
    lax.fori_loop(0, OWN + 16, zc, ())

    def scan_chunk(c, off):
        pltpu.sync_copy(dst_hbm.at[pl.ds(c * SCAN, SCAN)], dstv)

        def inner(k, off):
            d16 = dstv[pl.ds(k * 16, 16)]
            dl16 = d16 - lo
            m16 = jnp.where((dl16 >= 0) & (dl16 < OWN), 1, 0)
            for j in range(16):
                dlj = dl16[j]
                mj = m16[j]
                e = c * SCAN + k * 16 + j
                csr_eid[pl.ds(off, 16)] = z16 + e
                csr_dstl[pl.ds(off, 16)] = z16 + dlj
                ns = jnp.where(mj == 1, dlj, OWN + 8)
                counts[ns] = counts[ns] + mj
                off = jnp.minimum(off + mj, CAP - 1)
            return off
        return lax.fori_loop(0, SCAN // 16, inner, off)

    cnt = lax.fori_loop(0, NSCAN, scan_chunk, jnp.int32(0))

    npad = (CAP - cnt + 15) // 16
    def padf(k, _):
        csr_eid[pl.ds(cnt + k * 16, 16)] = z16
        csr_dstl[pl.ds(cnt + k * 16, 16)] = z16 + OWN
        return ()
    lax.fori_loop(0, npad, padf, ())

    def degcopy(n, _):
        degv[pl.ds(n, 16)] = z16 + counts[n]
        return ()
    lax.fori_loop(0, OWN, degcopy, ())

    pltpu.sync_copy(csr_eid.at[pl.ds(0, CAP)], eid_hbm.at[w])
    pltpu.sync_copy(csr_dstl.at[pl.ds(0, CAP)], dstl_hbm.at[w])
    pltpu.sync_copy(degv.at[pl.ds(0, OWN)], deg_hbm.at[pl.ds(w * OWN, OWN)])


_prep = pl.kernel(
    _prep_body,
    out_type=(jax.ShapeDtypeStruct((NW, CAP), jnp.int32),
              jax.ShapeDtypeStruct((NW, CAP), jnp.int32),
              jax.ShapeDtypeStruct((N_PAD,), jnp.int32)),
    mesh=_sc_mesh,
    scratch_types=[
        pltpu.VMEM((SCAN,), jnp.int32),
        pltpu.VMEM((CAP + 32,), jnp.int32),
        pltpu.VMEM((CAP + 32,), jnp.int32),
        pltpu.VMEM((OWN + 16,), jnp.int32),
        pltpu.SMEM((OWN + 16,), jnp.int32),
        pltpu.SemaphoreType.DMA,
    ],
    compiler_params=_sc_params,
)


def _gather2_body(h_hbm, dst_hbm, src_hbm, hd_hbm, hs_hbm,
                  idxd_v, idxs_v, rowsd_v, rowss_v, semd, sems):
    w = _wid()
    base = w * EPT
    pltpu.sync_copy(dst_hbm.at[pl.ds(base, EPT)], idxd_v)
    pltpu.sync_copy(src_hbm.at[pl.ds(base, EPT)], idxs_v)

    def chunk(off, sz):
        cpd = pltpu.async_copy(h_hbm.at[idxd_v.at[pl.ds(off, sz)]],
                               rowsd_v.at[pl.ds(0, sz)], semd)
        cps = pltpu.async_copy(h_hbm.at[idxs_v.at[pl.ds(off, sz)]],
                               rowss_v.at[pl.ds(0, sz)], sems)
        cpd.wait()
        cps.wait()
        pltpu.sync_copy(rowsd_v.at[pl.ds(0, sz)], hd_hbm.at[pl.ds(base + off, sz)])
        pltpu.sync_copy(rowss_v.at[pl.ds(0, sz)], hs_hbm.at[pl.ds(base + off, sz)])

    def body(i, _):
        chunk(i * CH, CH)
        return ()
    lax.fori_loop(0, GFULL, body, ())
    chunk(GFULL * CH, GREM)


_gather2 = pl.kernel(
    _gather2_body,
    out_type=(jax.ShapeDtypeStruct((N_EDGES, F), jnp.float32),
              jax.ShapeDtypeStruct((N_EDGES, F), jnp.float32)),
    mesh=_sc_mesh,
    scratch_types=[
        pltpu.VMEM((EPT,), jnp.int32),
        pltpu.VMEM((EPT,), jnp.int32),
        pltpu.VMEM((CH, F), jnp.float32),
        pltpu.VMEM((CH, F), jnp.float32),
        pltpu.SemaphoreType.DMA,
        pltpu.SemaphoreType.DMA,
    ],
    compiler_params=_sc_params,
)


def _reduce_body(msgs_hbm, eid_hbm, dstl_hbm, sum_hbm, sq_hbm, mn_hbm, mx_hbm,
                 eidv, idxbuf, dlv, stage, acc_sum, acc_sq, acc_mn, acc_mx, sem):
    w = _wid()
    pltpu.sync_copy(eid_hbm.at[w], eidv)
    pltpu.sync_copy(dstl_hbm.at[w], dlv.at[pl.ds(0, CAP)])
    zf = jnp.zeros((16,), jnp.float32)
    inf = jnp.full((16,), jnp.inf, jnp.float32)

    for t in range(T):

        def zacc(i, _):
            for j in range(F // 16):
                acc_sum[i, pl.ds(j * 16, 16)] = zf
                acc_sq[i, pl.ds(j * 16, 16)] = zf
                acc_mn[i, pl.ds(j * 16, 16)] = inf
                acc_mx[i, pl.ds(j * 16, 16)] = -inf
            return ()
        lax.fori_loop(0, OWN + 8, zacc, ())

        def chunk(c, _):
            for k in range(CH // 16):
                idxbuf[pl.ds(k * 16, 16)] = eidv[pl.ds(c * CH + k * 16, 16)] + t * N_EDGES
            pltpu.async_copy(msgs_hbm.at[idxbuf], stage, sem).wait()

            def edge(e, _):
                n = _sread(dlv, c * CH + e)
                for j in range(F // 16):
                    v = stage[e, pl.ds(j * 16, 16)]
                    plsc.addupdate(acc_sum.at[n, pl.ds(j * 16, 16)], v)
                    plsc.addupdate(acc_sq.at[n, pl.ds(j * 16, 16)], v * v)
                    m0 = acc_mn[n, pl.ds(j * 16, 16)]
                    acc_mn[n, pl.ds(j * 16, 16)] = jnp.minimum(m0, v)
                    m1 = acc_mx[n, pl.ds(j * 16, 16)]
                    acc_mx[n, pl.ds(j * 16, 16)] = jnp.maximum(m1, v)
                return ()
            lax.fori_loop(0, CH, edge, ())
            return ()
        lax.fori_loop(0, NCH, chunk, ())

        row0 = w * OWN
        pltpu.sync_copy(acc_sum.at[pl.ds(0, OWN)], sum_hbm.at[t, pl.ds(row0, OWN)])
        pltpu.sync_copy(acc_sq.at[pl.ds(0, OWN)], sq_hbm.at[t, pl.ds(row0, OWN)])
        pltpu.sync_copy(acc_mn.at[pl.ds(0, OWN)], mn_hbm.at[t, pl.ds(row0, OWN)])
        pltpu.sync_copy(acc_mx.at[pl.ds(0, OWN)], mx_hbm.at[t, pl.ds(row0, OWN)])


_reduce = pl.kernel(
    _reduce_body,
    out_type=(jax.ShapeDtypeStruct((T, N_PAD, F), jnp.float32),
              jax.ShapeDtypeStruct((T, N_PAD, F), jnp.float32),
              jax.ShapeDtypeStruct((T, N_PAD, F), jnp.float32),
              jax.ShapeDtypeStruct((T, N_PAD, F), jnp.float32)),
    mesh=_sc_mesh,
    scratch_types=[
        pltpu.VMEM((CAP,), jnp.int32),
        pltpu.VMEM((CH,), jnp.int32),
        pltpu.VMEM((CAP + 16,), jnp.int32),
        pltpu.VMEM((CH, F), jnp.float32),
        pltpu.VMEM((OWN + 8, F), jnp.float32),
        pltpu.VMEM((OWN + 8, F), jnp.float32),
        pltpu.VMEM((OWN + 8, F), jnp.float32),
        pltpu.VMEM((OWN + 8, F), jnp.float32),
        pltpu.SemaphoreType.DMA,
    ],
    compiler_params=_sc_params,
)


BE = 800


def _msgs_body(hd_ref, hs_ref, ea_ref, we_ref, bee_ref, wp_ref, bp_ref,
               out_ref, mcat):
    e80 = jnp.dot(ea_ref[...], we_ref[...], preferred_element_type=jnp.float32) + bee_ref[...]
    mcat[:, 0:F] = hd_ref[...]
    mcat[:, F:2 * F] = hs_ref[...]
    mcat[:, 2 * F:3 * F] = e80
    out_ref[...] = jnp.dot(mcat[...], wp_ref[0], preferred_element_type=jnp.float32) + bp_ref[0]


def _msgs_call(hd, hs, ea, We_l, be_l, Wp_l, bp_l):
    return pl.pallas_call(
        _msgs_body,
        grid=(T, N_EDGES // BE),
        in_specs=[pl.BlockSpec((BE, F), lambda t, i: (i, 0)),
                  pl.BlockSpec((BE, F), lambda t, i: (i, 0)),
                  pl.BlockSpec((BE, EDGE_DIM), lambda t, i: (i, 0)),
                  pl.BlockSpec((EDGE_DIM, F), lambda t, i: (0, 0)),
                  pl.BlockSpec((1, F), lambda t, i: (0, 0)),
                  pl.BlockSpec((1, 3 * F, F), lambda t, i: (t, 0, 0)),
                  pl.BlockSpec((1, 1, F), lambda t, i: (t, 0, 0))],
        out_specs=pl.BlockSpec((BE, F), lambda t, i: (t * (N_EDGES // BE) + i, 0)),
        out_shape=jax.ShapeDtypeStruct((T * N_EDGES, F), jnp.float32),
        scratch_shapes=[pltpu.VMEM((BE, 3 * F), jnp.float32)],
    )(hd, hs, ea, We_l, be_l.reshape(1, F), Wp_l, bp_l.reshape(T, 1, F))


BN_ = 400


def _postprep_body(sum_ref, sq_ref, mn_ref, mx_ref, degf_ref, h_ref, o_ref):
    degf = degf_ref[...]
    degc = jnp.maximum(degf, 1.0)
    logd = jnp.log(degc + 1.0)
    has_edge = degf > 0.0
    s_amp = logd / _AVG_DEG_LOG
    s_att = _AVG_DEG_LOG / logd
    for t in range(T):
        mean = sum_ref[t] / degc
        msq = sq_ref[t] / degc
        std = jnp.sqrt(jnp.maximum(msq - mean * mean, 0.0) + 1e-5)
        mn = jnp.where(has_edge, mn_ref[t], 0.0)
        mx = jnp.where(has_edge, mx_ref[t], 0.0)
        o_ref[:, t, 0:F] = h_ref[...]
        pieces = (mean, mn, mx, std)
        for p in range(4):
            o_ref[:, t, (1 + p) * F:(2 + p) * F] = pieces[p]
        for p in range(4):
            o_ref[:, t, (5 + p) * F:(6 + p) * F] = pieces[p] * s_amp
        for p in range(4):
            o_ref[:, t, (9 + p) * F:(10 + p) * F] = pieces[p] * s_att


def _postprep_call(ssum, ssq, smn, smx, degf, h):
    return pl.pallas_call(
        _postprep_body,
        grid=(N_NODES // BN_,),
        in_specs=[pl.BlockSpec((T, BN_, F), lambda i: (0, i, 0)),
                  pl.BlockSpec((T, BN_, F), lambda i: (0, i, 0)),
                  pl.BlockSpec((T, BN_, F), lambda i: (0, i, 0)),
                  pl.BlockSpec((T, BN_, F), lambda i: (0, i, 0)),
                  pl.BlockSpec((BN_, 1), lambda i: (i, 0)),
                  pl.BlockSpec((BN_, F), lambda i: (i, 0))],
        out_specs=pl.BlockSpec((BN_, T, 13 * F), lambda i: (i, 0, 0)),
        out_shape=jax.ShapeDtypeStruct((N_NODES, T, 13 * F), jnp.float32),
    )(ssum, ssq, smn, smx, degf, h)


def _postdot_body(o_ref, w_ref, b_ref, out_ref):
    for t in range(T):
        out_ref[:, t, :] = jnp.dot(o_ref[:, t, :], w_ref[t],
                                   preferred_element_type=jnp.float32) + b_ref[:, t, :]


def _postdot_call(o_cat, Wp, bp):
    return pl.pallas_call(
        _postdot_body,
        grid=(N_NODES // BN_,),
        in_specs=[pl.BlockSpec((BN_, T, 13 * F), lambda i: (i, 0, 0)),
                  pl.BlockSpec((T, 13 * F, F_OUT), lambda i: (0, 0, 0)),
                  pl.BlockSpec((1, T, F_OUT), lambda i: (0, 0, 0))],
        out_specs=pl.BlockSpec((BN_, T, F_OUT), lambda i: (i, 0, 0)),
        out_shape=jax.ShapeDtypeStruct((N_NODES, T, F_OUT), jnp.float32),
    )(o_cat, Wp, bp.reshape(1, T, F_OUT))


def _lin_body(o_ref, w_ref, b_ref, out_ref):
    out_ref[...] = jnp.dot(o_ref[...], w_ref[...],
                           preferred_element_type=jnp.float32) + b_ref[...]


def _lin_call(o, W, b):
    return pl.pallas_call(
        _lin_body,
        grid=(N_NODES // BN_,),
        in_specs=[pl.BlockSpec((BN_, F), lambda i: (i, 0)),
                  pl.BlockSpec((F, F), lambda i: (0, 0)),
                  pl.BlockSpec((1, F), lambda i: (0, 0))],
        out_specs=pl.BlockSpec((BN_, F), lambda i: (i, 0)),
        out_shape=jax.ShapeDtypeStruct((N_NODES, F), jnp.float32),
    )(o, W, b.reshape(1, F))


_NV = N_NODES // 8
_NACC = 16
_NIT = _NV // _NACC
_NREM = _NV - _NIT * _NACC


def _bn_reduce(load_slice):
    accs = lax.fori_loop(
        0, _NIT,
        lambda i, accs: [a + load_slice((i * _NACC + j) * 8) for j, a in enumerate(accs)],
        [jnp.zeros((8, F), jnp.float32) for _ in range(_NACC)])
    for r in range(_NREM):
        accs[r] = accs[r] + load_slice((_NIT * _NACC + r) * 8)
    s = accs[0]
    for a in accs[1:]:
        s = s + a
    b = s[0:4] + s[4:8]
    c = b[0:2] + b[2:4]
    return (c[0:1] + c[1:2]) * np.float32(1.0 / N_NODES)


def _bn_body(o_ref, g_ref, be_ref, h_ref):
    mu = _bn_reduce(lambda r0: o_ref[pl.ds(r0, 8), :])

    def dev(r0):
        d = o_ref[pl.ds(r0, 8), :] - mu
        return d * d
    var = _bn_reduce(dev)
    denom = jnp.sqrt(var + 1e-5)
    g = g_ref[...]
    be = be_ref[...]

    def norm(i, _):
        r0 = i * 1000
        o = o_ref[pl.ds(r0, 1000), :]
        h_ref[pl.ds(r0, 1000), :] = jnp.maximum((o - mu) / denom * g + be, 0.0)
        return ()
    lax.fori_loop(0, N_NODES // 1000, norm, ())


def _bn_call(o, gamma, beta):
    return pl.pallas_call(
        _bn_body,
        in_specs=[pl.BlockSpec((N_NODES, F), lambda: (0, 0)),
                  pl.BlockSpec((1, F), lambda: (0, 0)),
                  pl.BlockSpec((1, F), lambda: (0, 0))],
        out_specs=pl.BlockSpec((N_NODES, F), lambda: (0, 0)),
        out_shape=jax.ShapeDtypeStruct((N_NODES, F), jnp.float32),
    )(o, gamma.reshape(1, F), beta.reshape(1, F))


def _head_body(h_ref, b_ref, w1_ref, b1_ref, w2_ref, b2_ref, w3_ref, b3_ref,
               out_ref):
    gids = jax.lax.broadcasted_iota(jnp.int32, (N_NODES, N_GRAPHS), 1)
    onehot = jnp.where(b_ref[...] == gids, 1.0, 0.0)
    pooled = jax.lax.dot_general(onehot, h_ref[...],
                                 (((0,), (0,)), ((), ())),
                                 preferred_element_type=jnp.float32)
    z = jnp.maximum(jnp.dot(pooled, w1_ref[...], preferred_element_type=jnp.float32) + b1_ref[...], 0.0)
    z = jnp.maximum(jnp.dot(z, w2_ref[...], preferred_element_type=jnp.float32) + b2_ref[...], 0.0)
    out_ref[...] = jnp.dot(z, w3_ref[...], preferred_element_type=jnp.float32) + b3_ref[...]


def _head_call(h, batch, Wm1, bm1, Wm2, bm2, Wm3, bm3):
    return pl.pallas_call(
        _head_body,
        in_specs=[pl.BlockSpec((N_NODES, F), lambda: (0, 0)),
                  pl.BlockSpec((N_NODES, 1), lambda: (0, 0)),
                  pl.BlockSpec((F, 256), lambda: (0, 0)),
                  pl.BlockSpec((1, 256), lambda: (0, 0)),
                  pl.BlockSpec((256, 128), lambda: (0, 0)),
                  pl.BlockSpec((1, 128), lambda: (0, 0)),
                  pl.BlockSpec((128, 1), lambda: (0, 0)),
                  pl.BlockSpec((1, 1), lambda: (0, 0))],
        out_specs=pl.BlockSpec((N_GRAPHS, 1), lambda: (0, 0)),
        out_shape=jax.ShapeDtypeStruct((N_GRAPHS, 1), jnp.float32),
    )(h, batch.reshape(N_NODES, 1), Wm1, bm1.reshape(1, 256),
      Wm2, bm2.reshape(1, 128), Wm3, bm3.reshape(1, 1))


def kernel(x, edge_index, edge_attr, batch, We, be, Wpre, bpre, Wpost, bpost,
           Wlin, blin, gamma, beta, Wm1, bm1, Wm2, bm2, Wm3, bm3):
    src = edge_index[0]
    dst = edge_index[1]
    csr_eid, csr_dstl, deg_i = _prep(dst)
    degf = deg_i[:N_NODES].astype(jnp.float32).reshape(N_NODES, 1)
    h = x
    for l in range(N_LAYERS):
        hd, hs = _gather2(h, dst, src)
        msgs2d = _msgs_call(hd, hs, edge_attr, We[l], be[l], Wpre[l], bpre[l])
        ssum, ssq, smn, smx = _reduce(msgs2d, csr_eid, csr_dstl)
        o_cat = _postprep_call(ssum, ssq, smn, smx, degf, h)
        if l == 0:
            o = jnp.einsum('ntc,tcf->ntf', o_cat, Wpost[l]) + bpost[l]
        else:
            o = _postdot_call(o_cat, Wpost[l], bpost[l])
        o = o.reshape(N_NODES, T * F_OUT)
        o = _lin_call(o, Wlin[l], blin[l])
        h = _bn_call(o, gamma[l], beta[l])
    return _head_call(h, batch, Wm1, bm1, Wm2, bm2, Wm3, bm3)
